# Initial kernel scaffold; baseline (speedup 1.0000x reference)
import jax
import jax.numpy as jnp
from jax.experimental import pallas as pl


def kernel(adj_t, emb, W1, b1, W2, b2):
    raise NotImplementedError("write your pallas kernel here")


# speedup vs baseline: 15.7189x; 15.7189x over previous
import functools

import jax
import jax.numpy as jnp
from jax import lax
from jax.experimental import pallas as pl
from jax.experimental.pallas import tpu as pltpu
from jax.experimental.pallas import tpu_sc as plsc

N = 10000
D = 128
E = 320000
NR = 10240
NC, NS = 2, 16
NW = NC * NS
ET = E + N
NCHUNK = 81
EPT = NCHUNK * 128
EPAD = NW * EPT
ROWS_PER_TILE = NR // NS

_MESH = plsc.VectorSubcoreMesh(
    core_axis_name="c", subcore_axis_name="s", num_cores=NC, num_subcores=NS
)

f32 = jnp.float32


NSLOT = 16384


@functools.partial(
    pl.kernel,
    out_type=jax.ShapeDtypeStruct((NW, NSLOT), f32),
    mesh=_MESH,
    scratch_types=[
        pltpu.VMEM((EPT,), jnp.int32),
        pltpu.VMEM((NSLOT,), f32),
    ],
    compiler_params=pltpu.CompilerParams(needs_layout_passes=False),
)
def _sc_degree(dst_hbm, out_hbm, dst_v, hist_v):
    c = lax.axis_index("c")
    s = lax.axis_index("s")
    wid = c * NS + s
    pltpu.sync_copy(dst_hbm.at[wid], dst_v)

    zeros16 = jnp.zeros((16,), f32)

    def zstep(i, carry):
        hist_v[pl.ds(i * 16, 16)] = zeros16
        return carry

    lax.fori_loop(0, NSLOT // 16, zstep, 0)

    ones = jnp.full((16,), 1.0, f32)

    def step(i, carry):
        v = dst_v[pl.ds(i * 16, 16)]
        plsc.addupdate_scatter(hist_v, [v], ones)
        return carry

    lax.fori_loop(0, EPT // 16, step, 0)
    pltpu.sync_copy(hist_v, out_hbm.at[wid])


@functools.partial(
    pl.kernel,
    out_type=jax.ShapeDtypeStruct((NC, NR, D), f32),
    mesh=_MESH,
    scratch_types=[
        pltpu.VMEM((NCHUNK, 128), jnp.int32),
        pltpu.VMEM((NCHUNK, 128), jnp.int32),
        pltpu.VMEM((128, D), f32),
        pltpu.VMEM_SHARED((NR, D), f32),
        pltpu.SemaphoreType.DMA,
    ],
)
def _sc_edge_pass(src_hbm, dst_hbm, g_hbm, zeros_hbm, out_hbm,
                  src_v, dst_v, rows_v, acc_s, sem):
    c = lax.axis_index("c")
    s = lax.axis_index("s")
    wid = c * NS + s
    pltpu.sync_copy(src_hbm.at[wid], src_v)
    pltpu.sync_copy(dst_hbm.at[wid], dst_v)
    r0 = s * ROWS_PER_TILE
    pltpu.sync_copy(zeros_hbm.at[pl.ds(r0, ROWS_PER_TILE)],
                    acc_s.at[pl.ds(r0, ROWS_PER_TILE)])
    plsc.subcore_barrier()

    def step(j, carry):
        pltpu.async_copy(g_hbm.at[src_v.at[j]], rows_v, sem).wait()
        pltpu.sync_copy(rows_v, acc_s.at[dst_v.at[j]], add=True)
        return carry

    lax.fori_loop(0, NCHUNK, step, 0)

    plsc.subcore_barrier()
    pltpu.sync_copy(acc_s.at[pl.ds(r0, ROWS_PER_TILE)],
                    out_hbm.at[c, pl.ds(r0, ROWS_PER_TILE)])


_BS = 512
_G = NR // _BS


def _row_spec():
    return pl.BlockSpec((_BS, D), lambda i: (i, 0))


def _col_spec():
    return pl.BlockSpec((_BS, 1), lambda i: (i, 0))


def _tc_pre_body(degt_ref, emb_ref, w_ref, g_ref, dinv_ref):
    i = pl.program_id(0)
    deg = jnp.sum(degt_ref[...], axis=1, keepdims=True)
    rid = lax.broadcasted_iota(jnp.int32, (_BS, 1), 0) + i * _BS
    dinv = jnp.where(rid < N, lax.rsqrt(jnp.maximum(deg, 1e-12)), 0.0)
    h = jnp.dot(emb_ref[...], w_ref[...], preferred_element_type=f32)
    g_ref[...] = h * dinv
    dinv_ref[...] = dinv


def _tc_pre(degt, emb_pad, w1):
    return pl.pallas_call(
        _tc_pre_body,
        grid=(_G,),
        in_specs=[pl.BlockSpec((_BS, NW), lambda i: (i, 0)), _row_spec(),
                  pl.BlockSpec((D, D), lambda i: (0, 0))],
        out_specs=[_row_spec(), _col_spec()],
        out_shape=[jax.ShapeDtypeStruct((NR, D), f32),
                   jax.ShapeDtypeStruct((NR, 1), f32)],
    )(degt, emb_pad, w1)


def _tc_mid_body(a0_ref, a1_ref, dinv_ref, b_ref, w_ref, g_ref):
    dinv = dinv_ref[...]
    x = jnp.maximum((a0_ref[...] + a1_ref[...]) * dinv + b_ref[...], 0.0)
    g_ref[...] = jnp.dot(x, w_ref[...], preferred_element_type=f32) * dinv


def _tc_mid(a0, a1, dinv, b1, w2):
    return pl.pallas_call(
        _tc_mid_body,
        grid=(_G,),
        in_specs=[_row_spec(), _row_spec(), _col_spec(),
                  pl.BlockSpec((1, D), lambda i: (0, 0)),
                  pl.BlockSpec((D, D), lambda i: (0, 0))],
        out_specs=_row_spec(),
        out_shape=jax.ShapeDtypeStruct((NR, D), f32),
    )(a0, a1, dinv, b1, w2)


def _tc_post_body(a0_ref, a1_ref, dinv_ref, b_ref, out_ref):
    out_ref[...] = (a0_ref[...] + a1_ref[...]) * dinv_ref[...] + b_ref[...]


def _tc_post(a0, a1, dinv, b2):
    return pl.pallas_call(
        _tc_post_body,
        grid=(_G,),
        in_specs=[_row_spec(), _row_spec(), _col_spec(),
                  pl.BlockSpec((1, D), lambda i: (0, 0))],
        out_specs=_row_spec(),
        out_shape=jax.ShapeDtypeStruct((NR, D), f32),
    )(a0, a1, dinv, b2)


def kernel(adj_t, emb, W1, b1, W2, b2):
    loop = jnp.arange(N, dtype=jnp.int32)
    pad = jnp.full((EPAD - ET,), N, jnp.int32)
    src = jnp.concatenate([adj_t[0].astype(jnp.int32), loop, pad])
    dst = jnp.concatenate([adj_t[1].astype(jnp.int32), loop, pad])
    src_t = src.reshape(NW, NCHUNK, 128)
    dst_t = dst.reshape(NW, NCHUNK, 128)
    dst_flat = dst.reshape(NW, EPT)

    zeros = jnp.zeros((NR, D), f32)
    emb_pad = jnp.zeros((NR, D), f32).at[:N].set(emb)

    degp = _sc_degree(dst_flat)
    degt = degp.T[:NR]

    g1, dinv = _tc_pre(degt, emb_pad, W1)
    acc1 = _sc_edge_pass(src_t, dst_t, g1, zeros)
    g2 = _tc_mid(acc1[0], acc1[1], dinv, b1.reshape(1, D), W2)
    acc2 = _sc_edge_pass(src_t, dst_t, g2, zeros)
    out = _tc_post(acc2[0], acc2[1], dinv, b2.reshape(1, D))
    return out[:N]
